# Initial kernel scaffold; baseline (speedup 1.0000x reference)
import jax
import jax.numpy as jnp
from jax.experimental import pallas as pl


def kernel(input_ids, tok_emb, pos_emb):
    raise NotImplementedError("write your pallas kernel here")


# speedup vs baseline: 1.0025x; 1.0025x over previous
import functools

import jax
import jax.numpy as jnp
from jax import lax
from jax.experimental import pallas as pl
from jax.experimental.pallas import tpu as pltpu
from jax.experimental.pallas import tpu_sc as plsc

VOCAB = 100000
D = 768
B = 4
T = 2048

_INFO = plsc.get_sparse_core_info()
NC, NS, L = _INFO.num_cores, _INFO.num_subcores, _INFO.num_lanes
NW = NC * NS

ROWS = B * T
ROWS_PER_W = ROWS // NW
CHUNK = 32
N_CHUNKS = ROWS_PER_W // CHUNK


def _body(ids_hbm, pos_hbm, tok_hbm, out_hbm, idx_v, rows_v, pos_v, sem):
    wid = lax.axis_index("c") * NS + lax.axis_index("s")
    base = wid * ROWS_PER_W
    t0 = base % T

    def chunk_step(c, _):
        rbase = base + c * CHUNK
        pbase = t0 + c * CHUNK
        pltpu.sync_copy(ids_hbm.at[pl.ds(rbase, CHUNK)], idx_v)
        gather = pltpu.async_copy(tok_hbm.at[idx_v], rows_v, sem)
        pltpu.sync_copy(pos_hbm.at[pl.ds(pbase, CHUNK)], pos_v)
        gather.wait()

        def row_step(r, _):
            for g in range(D // L):
                sl = pl.ds(g * L, L)
                rows_v[r, sl] = rows_v[r, sl] + pos_v[r, sl]
            return 0

        lax.fori_loop(0, CHUNK, row_step, 0)
        pltpu.sync_copy(rows_v, out_hbm.at[pl.ds(rbase, CHUNK)])
        return 0

    lax.fori_loop(0, N_CHUNKS, chunk_step, 0)


@jax.jit
def _embed(ids_flat, tok_emb, pos_emb):
    mesh = plsc.VectorSubcoreMesh(core_axis_name="c", subcore_axis_name="s")
    k = functools.partial(
        pl.kernel,
        mesh=mesh,
        out_type=jax.ShapeDtypeStruct((ROWS, D), jnp.float32),
        scratch_types=[
            pltpu.VMEM((CHUNK,), jnp.int32),
            pltpu.VMEM((CHUNK, D), jnp.float32),
            pltpu.VMEM((CHUNK, D), jnp.float32),
            pltpu.SemaphoreType.DMA,
        ],
    )(_body)
    return k(ids_flat, pos_emb, tok_emb)


def kernel(input_ids, tok_emb, pos_emb):
    ids_flat = input_ids.reshape(-1).astype(jnp.int32)
    out = _embed(ids_flat, tok_emb, pos_emb)
    return out.reshape(B, T, D)
